# baseline (speedup 1.0000x reference)
import jax
import jax.numpy as jnp
from jax import lax
from jax.experimental import pallas as pl
from jax.experimental.pallas import tpu as pltpu
from jax.experimental.pallas import tpu_sc as plsc

NQ = 50000
NW = 50000
V = 100000
E = 800000
B = 128
D_IN = 50
D_HID = 32
D_OUT = 32

NC = 2
NS = 16
NWK = NC * NS

CH = 128
ROWS_E = E // CH
RPW = ROWS_E // NWK
TAILR = RPW * NWK
NJ = 200
NBUF = 8
NQ2 = 51200
DUMP_DST = 50048
DUMP_SRC = 50560

G_ROWS = 416
GPW = G_ROWS // NWK
NQG = G_ROWS * CH

STRIPE = NQ // NS

_mesh = plsc.VectorSubcoreMesh(core_axis_name="c", subcore_axis_name="s")
_sc_params = pltpu.CompilerParams(use_tc_tiling_on_sc=False)


def _dummy_row(idxs, plane, r, base):
    for t in range(CH // 16):
        idxs[plane, r, pl.ds(16 * t, 16)] = (
            base + 16 * t + lax.iota(jnp.int32, 16))


def _sc_pre_body(es, eu, wid, we,
                 g1_out, cnt_out,
                 idxs, gidx, rows_v, ones_v, zb1, sem, sem_s,
                 c_os, c_is, c_ou, c_iu):
    c = lax.axis_index("c")
    s = lax.axis_index("s")
    w = s * NC + c
    accs = (c_os, c_is, c_ou, c_iu)

    def _init_ones(i, _):
        ones_v[pl.ds(i * 16, 16)] = jnp.ones((16,), jnp.float32)
        return 0
    lax.fori_loop(0, CH // 16, _init_ones, 0)

    def _init_zb1(i, _):
        zb1[pl.ds(i * 16, 16)] = jnp.zeros((16,), jnp.float32)
        return 0
    lax.fori_loop(0, 3200 // 16, _init_zb1, 0)

    for acc in accs:
        pltpu.sync_copy(zb1, acc.at[pl.ds(s * 3200, 3200)])
    plsc.subcore_barrier()

    for edges, acc_src, acc_dst in ((es, c_os, c_is), (eu, c_ou, c_iu)):
        for k in (0, 1):
            pltpu.sync_copy(edges.at[k, pl.ds(w * RPW, RPW)],
                            idxs.at[k, pl.ds(0, RPW)])

        @pl.when(w < ROWS_E - TAILR)
        def _(edges=edges):
            for k in (0, 1):
                pltpu.sync_copy(edges.at[k, TAILR + w], idxs.at[k, RPW])

        @pl.when(w >= ROWS_E - TAILR)
        def _():
            _dummy_row(idxs, 0, RPW, DUMP_SRC)
            _dummy_row(idxs, 1, RPW, DUMP_DST)

        def _grp(g, _, acc_src=acc_src, acc_dst=acc_dst):
            for b in range(4):
                j = g * 4 + b
                pltpu.async_copy(ones_v, acc_src.at[idxs.at[0, j]], sem_s,
                                 add=True)
                pltpu.async_copy(ones_v, acc_dst.at[idxs.at[1, j]], sem_s,
                                 add=True)
            for b in range(4):
                j = g * 4 + b
                pltpu.make_async_copy(ones_v, acc_src.at[idxs.at[0, j]],
                                      sem_s).wait()
                pltpu.make_async_copy(ones_v, acc_dst.at[idxs.at[1, j]],
                                      sem_s).wait()
            return 0
        lax.fori_loop(0, (RPW + 1) // 4, _grp, 0)

    pltpu.sync_copy(wid.at[pl.ds(w * GPW, GPW)], gidx)

    def _gbody(j, _):
        base = (w * GPW + j) * CH
        pltpu.async_copy(we.at[gidx.at[j]], rows_v, sem).wait()
        pltpu.sync_copy(rows_v, g1_out.at[pl.ds(base, CH)])
        return 0
    lax.fori_loop(0, GPW, _gbody, 0)

    plsc.subcore_barrier()
    for k, acc in enumerate(accs):
        @pl.when(s < NS - 1)
        def _(k=k, acc=acc):
            pltpu.sync_copy(acc.at[pl.ds(s * 3200, 3200)], zb1)
            pltpu.sync_copy(zb1, cnt_out.at[c, k, pl.ds(s * 3200, 3200)])

        @pl.when(s == NS - 1)
        def _(k=k, acc=acc):
            pltpu.sync_copy(acc.at[pl.ds(48000, 2000)], zb1.at[pl.ds(0, 2000)])
            pltpu.sync_copy(zb1.at[pl.ds(0, 2000)],
                            cnt_out.at[c, k, pl.ds(48000, 2000)])


def _sc_pre(es, eu, wid, we, name):
    return pl.kernel(
        _sc_pre_body,
        out_type=(
            jax.ShapeDtypeStruct((NQG, D_HID), jnp.bfloat16),
            jax.ShapeDtypeStruct((NC, 4, NQ), jnp.float32),
        ),
        mesh=_mesh,
        compiler_params=_sc_params,
        scratch_types=[
            pltpu.VMEM((2, RPW + 1, CH), jnp.int32),
            pltpu.VMEM((GPW, CH), jnp.int32),
            pltpu.VMEM((CH, D_HID), jnp.bfloat16),
            pltpu.VMEM((CH,), jnp.float32),
            pltpu.VMEM((3200,), jnp.float32),
            pltpu.SemaphoreType.DMA,
            pltpu.SemaphoreType.DMA,
        ] + [pltpu.VMEM_SHARED((NQ2,), jnp.float32)] * 4,
        name=name,
    )(es, eu, wid, we)


def _sc_edge_body(y, edges, out,
                  idxs, rows, zb, bb, sem_g, sem_s, acc):
    c = lax.axis_index("c")
    s = lax.axis_index("s")
    w = s * NC + c

    def _init_zb(i, _):
        zb[i, :] = jnp.zeros((D_HID,), jnp.bfloat16)
        return 0
    lax.fori_loop(0, CH, _init_zb, 0)

    for k in (0, 1):
        pltpu.sync_copy(edges.at[k, pl.ds(w * RPW, RPW)],
                        idxs.at[k, pl.ds(0, RPW)])

    @pl.when(w < ROWS_E - TAILR)
    def _():
        for k in (0, 1):
            pltpu.sync_copy(edges.at[k, TAILR + w], idxs.at[k, RPW])

    @pl.when(w >= ROWS_E - TAILR)
    def _():
        _dummy_row(idxs, 0, RPW, 0)
        _dummy_row(idxs, 1, RPW, DUMP_DST)
    for r in range(RPW + 1, NJ):
        _dummy_row(idxs, 0, r, 0)
        _dummy_row(idxs, 1, r, DUMP_DST)

    def _zero_chunk(k, _):
        pltpu.sync_copy(zb, acc.at[pl.ds(s * 3200 + k * CH, CH)])
        return 0
    lax.fori_loop(0, 3200 // CH, _zero_chunk, 0)
    plsc.subcore_barrier()

    for j in range(4):
        pltpu.async_copy(y.at[idxs.at[0, j]], rows[j], sem_g[j])

    def _grp(g, _):
        for b in range(NBUF):
            j = g * NBUF + b
            pltpu.make_async_copy(y.at[idxs.at[0, j]], rows[b],
                                  sem_g[b]).wait()
            pltpu.async_copy(rows[b], acc.at[idxs.at[1, j]], sem_s[b],
                             add=True)
            jn = j + 4
            bn = (b + 4) % NBUF

            @pl.when(jn < NJ)
            def _(jn=jn, bn=bn):
                @pl.when(jn >= NBUF)
                def _():
                    pltpu.make_async_copy(
                        rows[bn], acc.at[idxs.at[1, jn - NBUF]],
                        sem_s[bn]).wait()
                pltpu.async_copy(y.at[idxs.at[0, jn]], rows[bn],
                                 sem_g[bn])
        return 0
    lax.fori_loop(0, NJ // NBUF, _grp, 0)

    for j in range(NJ - NBUF, NJ):
        b = j % NBUF
        pltpu.make_async_copy(rows[b], acc.at[idxs.at[1, j]],
                              sem_s[b]).wait()

    plsc.subcore_barrier()

    def _wb_chunk(k, _):
        base = s * STRIPE + k * 125
        pltpu.sync_copy(acc.at[pl.ds(base, 125)], bb)
        pltpu.sync_copy(bb, out.at[c, pl.ds(base, 125)])
        return 0
    lax.fori_loop(0, STRIPE // 125, _wb_chunk, 0)


def _sc_edge(y, ed, name):
    return pl.kernel(
        _sc_edge_body,
        out_type=jax.ShapeDtypeStruct((NC, NQ, D_HID), jnp.bfloat16),
        mesh=_mesh,
        compiler_params=_sc_params,
        scratch_types=[
            pltpu.VMEM((2, NJ, CH), jnp.int32),
            [pltpu.VMEM((CH, D_HID), jnp.bfloat16) for _ in range(NBUF)],
            pltpu.VMEM((CH, D_HID), jnp.bfloat16),
            pltpu.VMEM((125, D_HID), jnp.bfloat16),
            [pltpu.SemaphoreType.DMA for _ in range(NBUF)],
            [pltpu.SemaphoreType.DMA for _ in range(NBUF)],
            pltpu.VMEM_SHARED((NQ2, D_HID), jnp.bfloat16),
        ],
        name=name,
    )(y, ed)


RB = 2000
NRB = NQ // RB


def _we_matmul_body(we_ref, w1_ref, out_ref):
    out_ref[...] = jnp.dot(we_ref[...], w1_ref[...],
                           preferred_element_type=jnp.float32
                           ).astype(jnp.bfloat16)


def _we_matmul(wordemb, W1_side):
    blk = 2000
    return pl.pallas_call(
        _we_matmul_body,
        grid=(V // blk,),
        in_specs=[
            pl.BlockSpec((blk, D_IN), lambda i: (i, 0)),
            pl.BlockSpec((D_IN, D_HID), lambda i: (0, 0)),
        ],
        out_specs=pl.BlockSpec((blk, D_HID), lambda i: (i, 0)),
        out_shape=jax.ShapeDtypeStruct((V, D_HID), jnp.bfloat16),
        name="tc_we_matmul",
    )(wordemb, W1_side)


def _scales_body(cnt_ref, sc_ref):
    cnt = cnt_ref[...]
    sc4 = lax.rsqrt(jnp.maximum(cnt[0] + cnt[1], 1.0))
    sc_ref[...] = jnp.transpose(sc4, (1, 0))


def _scales(cnt, name):
    return pl.pallas_call(
        _scales_body,
        in_specs=[pl.BlockSpec((NC, 4, NQ), lambda: (0, 0, 0))],
        out_specs=pl.BlockSpec((NQ, 4), lambda: (0, 0)),
        out_shape=jax.ShapeDtypeStruct((NQ, 4), jnp.float32),
        name=name,
    )(cnt)


def _scale_body(g1_ref, sc_ref, y_ref):
    i = pl.program_id(0)
    sc0 = sc_ref[...][:, 0:1]
    y_ref[...] = (g1_ref[...].astype(jnp.float32) * sc0).astype(jnp.bfloat16)


def _scale(g1, scales, name):
    return pl.pallas_call(
        _scale_body,
        grid=(NRB,),
        in_specs=[
            pl.BlockSpec((RB, D_HID), lambda i: (i, 0)),
            pl.BlockSpec((RB, 4), lambda i: (i, 0)),
        ],
        out_specs=pl.BlockSpec((RB, D_HID), lambda i: (i, 0)),
        out_shape=jax.ShapeDtypeStruct((NQ, D_HID), jnp.bfloat16),
        name=name,
    )(g1, scales)


def _mid_body(ag_ref, sc_ref, b1s_ref, b1u_ref, w2_ref, y_ref):
    sc = sc_ref[...]
    b = (b1s_ref[...] + b1u_ref[...])[None, :]
    w2 = w2_ref[...]
    ag = (ag_ref[0].astype(jnp.float32) + ag_ref[1].astype(jnp.float32))
    h1 = jnp.maximum(ag * sc[:, 1:2] + b, 0.0)
    y2 = jnp.dot(h1, w2, preferred_element_type=jnp.float32)
    y_ref[...] = (y2 * sc[:, 2:3]).astype(jnp.bfloat16)


def _mid(agg, scales, b1_side, b1_upd, W2_upd, name):
    return pl.pallas_call(
        _mid_body,
        grid=(NRB,),
        in_specs=[
            pl.BlockSpec((NC, RB, D_HID), lambda i: (0, i, 0)),
            pl.BlockSpec((RB, 4), lambda i: (i, 0)),
            pl.BlockSpec((D_HID,), lambda i: (0,)),
            pl.BlockSpec((D_HID,), lambda i: (0,)),
            pl.BlockSpec((D_HID, D_OUT), lambda i: (0, 0)),
        ],
        out_specs=pl.BlockSpec((RB, D_HID), lambda i: (i, 0)),
        out_shape=jax.ShapeDtypeStruct((NQ, D_HID), jnp.bfloat16),
        name=name,
    )(agg, scales, b1_side, b1_upd, W2_upd)


def _seg_body(ag_ref, sc_ref, gid_ref, b2_ref, gsum_ref, gcnt_ref, acc, cnt):
    i = pl.program_id(0)

    @pl.when(i == 0)
    def _():
        acc[...] = jnp.zeros((B, D_OUT), jnp.float32)
        cnt[...] = jnp.zeros((B, 1), jnp.float32)

    b2 = b2_ref[...][None, :]
    iota = lax.broadcasted_iota(jnp.int32, (B, RB), 0)
    ones_col = jnp.ones((RB, 1), jnp.float32)
    ag = (ag_ref[0].astype(jnp.float32) + ag_ref[1].astype(jnp.float32))
    h2 = ag * sc_ref[...][:, 3:4] + b2
    gid = gid_ref[0, 0]
    mask = (gid[None, :] == iota).astype(jnp.float32)
    acc[...] += jnp.dot(mask, h2, preferred_element_type=jnp.float32)
    cnt[...] += jnp.dot(mask, ones_col, preferred_element_type=jnp.float32)

    @pl.when(i == NRB - 1)
    def _():
        gsum_ref[...] = acc[...]
        gcnt_ref[...] = cnt[...]


def _seg(agg2, scales, gid, b2_upd, name):
    gid3 = gid.reshape(NRB, 1, RB)
    return pl.pallas_call(
        _seg_body,
        grid=(NRB,),
        in_specs=[
            pl.BlockSpec((NC, RB, D_HID), lambda i: (0, i, 0)),
            pl.BlockSpec((RB, 4), lambda i: (i, 0)),
            pl.BlockSpec((1, 1, RB), lambda i: (i, 0, 0)),
            pl.BlockSpec((D_OUT,), lambda i: (0,)),
        ],
        out_specs=[
            pl.BlockSpec((B, D_OUT), lambda i: (0, 0)),
            pl.BlockSpec((B, 1), lambda i: (0, 0)),
        ],
        out_shape=[
            jax.ShapeDtypeStruct((B, D_OUT), jnp.float32),
            jax.ShapeDtypeStruct((B, 1), jnp.float32),
        ],
        scratch_shapes=[
            pltpu.VMEM((B, D_OUT), jnp.float32),
            pltpu.VMEM((B, 1), jnp.float32),
        ],
        name=name,
    )(agg2, scales, gid3, b2_upd)


def _loss_body(gsa_ref, gca_ref, gsp_ref, gcp_ref,
               click_ref, W0_ref, b0_ref, W1_ref, b1_ref, out_ref):
    anchor = gsa_ref[...] / jnp.maximum(gca_ref[...], 1.0)
    pos = gsp_ref[...] / jnp.maximum(gcp_ref[...], 1.0)
    W0a = W0_ref[:D_OUT, :]
    W0b = W0_ref[D_OUT:, :]
    b0 = b0_ref[...][None, :]
    W1 = W1_ref[...]
    b1 = b1_ref[...][None, :]
    z1 = jnp.maximum(anchor @ W0a + pos @ W0b + b0, 0.0)
    s1 = jax.nn.sigmoid(z1 @ W1 + b1)
    Aa = anchor @ W0a
    Ab = anchor @ W0b
    z2 = jnp.maximum(Aa[:, None, :] + Ab[None, :, :] + b0[None, :, :], 0.0)
    s2_ = jax.nn.sigmoid(
        z2.reshape(B * B, D_OUT) @ W1 + b1).reshape(B, B)
    res = s2_ * click_ref[...].astype(jnp.float32)
    s2 = jnp.max(res, axis=1)
    eps = 1e-12
    p1 = jnp.clip(s1[:, 0], eps, 1.0 - eps)
    p2 = jnp.clip(s2, eps, 1.0 - eps)
    loss = -jnp.mean(jnp.log(p1)) - jnp.mean(jnp.log(1.0 - p2))
    out_ref[...] = jnp.reshape(loss, (1, 1))


def _loss(gsa, gca, gsp, gcp, click, D_W0, D_b0, D_W1, D_b1):
    return pl.pallas_call(
        _loss_body,
        in_specs=[
            pl.BlockSpec((B, D_OUT), lambda: (0, 0)),
            pl.BlockSpec((B, 1), lambda: (0, 0)),
            pl.BlockSpec((B, D_OUT), lambda: (0, 0)),
            pl.BlockSpec((B, 1), lambda: (0, 0)),
            pl.BlockSpec((B, B), lambda: (0, 0)),
            pl.BlockSpec((2 * D_OUT, 32), lambda: (0, 0)),
            pl.BlockSpec((32,), lambda: (0,)),
            pl.BlockSpec((32, 1), lambda: (0, 0)),
            pl.BlockSpec((1,), lambda: (0,)),
        ],
        out_specs=pl.BlockSpec((1, 1), lambda: (0, 0)),
        out_shape=jax.ShapeDtypeStruct((1, 1), jnp.float32),
        name="tc_loss",
    )(gsa, gca, gsp, gcp, click, D_W0, D_b0, D_W1, D_b1)


def kernel(word_id_a, img_emb_a, edge_side_a, edge_upd_a, query_gid_a, word_id_p, img_emb_p, edge_side_p, edge_upd_p, query_gid_p, click_reverse, wordemb, trans_W, trans_b, W1_side, b1_side, W1_upd, b1_upd, W2_upd, b2_upd, D_W0, D_b0, D_W1, D_b1):
    pad_ids = jnp.arange(NQG - NQ, dtype=jnp.int32) % V
    wid_a = jnp.concatenate([word_id_a, pad_ids]).reshape(G_ROWS, CH)
    wid_p = jnp.concatenate([word_id_p, pad_ids]).reshape(G_ROWS, CH)
    es_a = edge_side_a.reshape(2, ROWS_E, CH)
    eu_a = edge_upd_a.reshape(2, ROWS_E, CH)
    es_p = edge_side_p.reshape(2, ROWS_E, CH)
    eu_p = edge_upd_p.reshape(2, ROWS_E, CH)

    we = _we_matmul(wordemb.astype(jnp.bfloat16),
                    W1_side.astype(jnp.bfloat16))
    g1a, cnt_a = _sc_pre(es_a, eu_a, wid_a, we, "sc_pre_a")
    g1p, cnt_p = _sc_pre(es_p, eu_p, wid_p, we, "sc_pre_p")
    sc_a = _scales(cnt_a, "tc_scales_a")
    sc_p = _scales(cnt_p, "tc_scales_p")
    ya = _scale(g1a, sc_a, "tc_scale_a")
    yp = _scale(g1p, sc_p, "tc_scale_p")
    agg_a = _sc_edge(ya, es_a, "sc_edge1_a")
    agg_p = _sc_edge(yp, es_p, "sc_edge1_p")
    y2a = _mid(agg_a, sc_a, b1_side, b1_upd, W2_upd, "tc_mid_a")
    y2p = _mid(agg_p, sc_p, b1_side, b1_upd, W2_upd, "tc_mid_p")
    agg2_a = _sc_edge(y2a, eu_a, "sc_edge2_a")
    agg2_p = _sc_edge(y2p, eu_p, "sc_edge2_p")
    gsa, gca = _seg(agg2_a, sc_a, query_gid_a, b2_upd, "tc_seg_a")
    gsp, gcp = _seg(agg2_p, sc_p, query_gid_p, b2_upd, "tc_seg_p")
    loss = _loss(gsa, gca, gsp, gcp, click_reverse, D_W0, D_b0, D_W1, D_b1)
    return jnp.reshape(loss, ())


# speedup vs baseline: 1.0228x; 1.0228x over previous
import jax
import jax.numpy as jnp
from jax import lax
from jax.experimental import pallas as pl
from jax.experimental.pallas import tpu as pltpu
from jax.experimental.pallas import tpu_sc as plsc

NQ = 50000
NW = 50000
V = 100000
E = 800000
B = 128
D_IN = 50
D_HID = 32
D_OUT = 32

NC = 2
NS = 16
NWK = NC * NS

CH = 128
ROWS_E = E // CH
RPW = ROWS_E // NWK
TAILR = RPW * NWK
NJ = 200
NBUF = 8
NQ2 = 51200
DUMP_DST = 50048
DUMP_SRC = 50560

G_ROWS = 416
GPW = G_ROWS // NWK
NQG = G_ROWS * CH

STRIPE = NQ // NS

_mesh = plsc.VectorSubcoreMesh(core_axis_name="c", subcore_axis_name="s")
_sc_params = pltpu.CompilerParams(use_tc_tiling_on_sc=False)


def _dummy_row(idxs, plane, r, base):
    for t in range(CH // 16):
        idxs[plane, r, pl.ds(16 * t, 16)] = (
            base + 16 * t + lax.iota(jnp.int32, 16))


def _sc_pre_body(es, eu, wid, we,
                 g1_out, cnt_out,
                 idxs, gidx, rows_v, ones_v, zb1, sem, sem_s,
                 c_os, c_is, c_ou, c_iu):
    c = lax.axis_index("c")
    s = lax.axis_index("s")
    w = s * NC + c
    accs = (c_os, c_is, c_ou, c_iu)

    def _init_ones(i, _):
        ones_v[pl.ds(i * 16, 16)] = jnp.ones((16,), jnp.float32)
        return 0
    lax.fori_loop(0, CH // 16, _init_ones, 0)

    def _init_zb1(i, _):
        zb1[pl.ds(i * 16, 16)] = jnp.zeros((16,), jnp.float32)
        return 0
    lax.fori_loop(0, 3200 // 16, _init_zb1, 0)

    for acc in accs:
        pltpu.sync_copy(zb1, acc.at[pl.ds(s * 3200, 3200)])
    plsc.subcore_barrier()

    for edges, acc_src, acc_dst in ((es, c_os, c_is), (eu, c_ou, c_iu)):
        for k in (0, 1):
            pltpu.sync_copy(edges.at[k, pl.ds(w * RPW, RPW)],
                            idxs.at[k, pl.ds(0, RPW)])

        @pl.when(w < ROWS_E - TAILR)
        def _(edges=edges):
            for k in (0, 1):
                pltpu.sync_copy(edges.at[k, TAILR + w], idxs.at[k, RPW])

        @pl.when(w >= ROWS_E - TAILR)
        def _():
            _dummy_row(idxs, 0, RPW, DUMP_SRC)
            _dummy_row(idxs, 1, RPW, DUMP_DST)

        def _grp(g, _, acc_src=acc_src, acc_dst=acc_dst):
            for b in range(4):
                j = g * 4 + b
                pltpu.async_copy(ones_v, acc_src.at[idxs.at[0, j]], sem_s,
                                 add=True)
                pltpu.async_copy(ones_v, acc_dst.at[idxs.at[1, j]], sem_s,
                                 add=True)
            for b in range(4):
                j = g * 4 + b
                pltpu.make_async_copy(ones_v, acc_src.at[idxs.at[0, j]],
                                      sem_s).wait()
                pltpu.make_async_copy(ones_v, acc_dst.at[idxs.at[1, j]],
                                      sem_s).wait()
            return 0
        lax.fori_loop(0, (RPW + 1) // 4, _grp, 0)

    pltpu.sync_copy(wid.at[pl.ds(w * GPW, GPW)], gidx)

    def _gbody(j, _):
        base = (w * GPW + j) * CH
        pltpu.async_copy(we.at[gidx.at[j]], rows_v, sem).wait()
        pltpu.sync_copy(rows_v, g1_out.at[pl.ds(base, CH)])
        return 0
    lax.fori_loop(0, GPW, _gbody, 0)

    plsc.subcore_barrier()
    for k, acc in enumerate(accs):
        @pl.when(s < NS - 1)
        def _(k=k, acc=acc):
            pltpu.sync_copy(acc.at[pl.ds(s * 3200, 3200)], zb1)
            pltpu.sync_copy(zb1, cnt_out.at[c, k, pl.ds(s * 3200, 3200)])

        @pl.when(s == NS - 1)
        def _(k=k, acc=acc):
            pltpu.sync_copy(acc.at[pl.ds(48000, 2000)], zb1.at[pl.ds(0, 2000)])
            pltpu.sync_copy(zb1.at[pl.ds(0, 2000)],
                            cnt_out.at[c, k, pl.ds(48000, 2000)])


def _sc_pre(es, eu, wid, we, name):
    return pl.kernel(
        _sc_pre_body,
        out_type=(
            jax.ShapeDtypeStruct((NQG, D_HID), jnp.bfloat16),
            jax.ShapeDtypeStruct((NC, 4, NQ), jnp.float32),
        ),
        mesh=_mesh,
        compiler_params=_sc_params,
        scratch_types=[
            pltpu.VMEM((2, RPW + 1, CH), jnp.int32),
            pltpu.VMEM((GPW, CH), jnp.int32),
            pltpu.VMEM((CH, D_HID), jnp.bfloat16),
            pltpu.VMEM((CH,), jnp.float32),
            pltpu.VMEM((3200,), jnp.float32),
            pltpu.SemaphoreType.DMA,
            pltpu.SemaphoreType.DMA,
        ] + [pltpu.VMEM_SHARED((NQ2,), jnp.float32)] * 4,
        name=name,
    )(es, eu, wid, we)


def _sc_edge_body(y, edges, out,
                  idxs, rows, zb, bb, sem_g, sem_s, acc):
    c = lax.axis_index("c")
    s = lax.axis_index("s")
    w = s * NC + c

    def _init_zb(i, _):
        zb[i, :] = jnp.zeros((D_HID,), jnp.bfloat16)
        return 0
    lax.fori_loop(0, CH, _init_zb, 0)

    for k in (0, 1):
        pltpu.sync_copy(edges.at[k, pl.ds(w * RPW, RPW)],
                        idxs.at[k, pl.ds(0, RPW)])

    @pl.when(w < ROWS_E - TAILR)
    def _():
        for k in (0, 1):
            pltpu.sync_copy(edges.at[k, TAILR + w], idxs.at[k, RPW])

    @pl.when(w >= ROWS_E - TAILR)
    def _():
        _dummy_row(idxs, 0, RPW, 0)
        _dummy_row(idxs, 1, RPW, DUMP_DST)
    for r in range(RPW + 1, NJ):
        _dummy_row(idxs, 0, r, 0)
        _dummy_row(idxs, 1, r, DUMP_DST)

    def _zero_chunk(k, _):
        pltpu.sync_copy(zb, acc.at[pl.ds(s * 3200 + k * CH, CH)])
        return 0
    lax.fori_loop(0, 3200 // CH, _zero_chunk, 0)
    plsc.subcore_barrier()

    for j in range(4):
        pltpu.async_copy(y.at[idxs.at[0, j]], rows[j], sem_g[j])

    def _grp(g, _):
        for b in range(NBUF):
            j = g * NBUF + b
            pltpu.make_async_copy(y.at[idxs.at[0, j]], rows[b],
                                  sem_g[b]).wait()
            pltpu.async_copy(rows[b], acc.at[idxs.at[1, j]], sem_s[b],
                             add=True)
            jn = j + 4
            bn = (b + 4) % NBUF

            @pl.when(jn < NJ)
            def _(jn=jn, bn=bn):
                @pl.when(jn >= NBUF)
                def _():
                    pltpu.make_async_copy(
                        rows[bn], acc.at[idxs.at[1, jn - NBUF]],
                        sem_s[bn]).wait()
                pltpu.async_copy(y.at[idxs.at[0, jn]], rows[bn],
                                 sem_g[bn])
        return 0
    lax.fori_loop(0, NJ // NBUF, _grp, 0)

    for j in range(NJ - NBUF, NJ):
        b = j % NBUF
        pltpu.make_async_copy(rows[b], acc.at[idxs.at[1, j]],
                              sem_s[b]).wait()

    plsc.subcore_barrier()

    def _wb_chunk(k, _):
        base = s * STRIPE + k * 125
        pltpu.sync_copy(acc.at[pl.ds(base, 125)], bb)
        pltpu.sync_copy(bb, out.at[c, pl.ds(base, 125)])
        return 0
    lax.fori_loop(0, STRIPE // 125, _wb_chunk, 0)


def _sc_edge(y, ed, name):
    return pl.kernel(
        _sc_edge_body,
        out_type=jax.ShapeDtypeStruct((NC, NQ, D_HID), jnp.bfloat16),
        mesh=_mesh,
        compiler_params=_sc_params,
        scratch_types=[
            pltpu.VMEM((2, NJ, CH), jnp.int32),
            [pltpu.VMEM((CH, D_HID), jnp.bfloat16) for _ in range(NBUF)],
            pltpu.VMEM((CH, D_HID), jnp.bfloat16),
            pltpu.VMEM((125, D_HID), jnp.bfloat16),
            [pltpu.SemaphoreType.DMA for _ in range(NBUF)],
            [pltpu.SemaphoreType.DMA for _ in range(NBUF)],
            pltpu.VMEM_SHARED((NQ2, D_HID), jnp.bfloat16),
        ],
        name=name,
    )(y, ed)


RB = 2000
NRB = NQ // RB


def _we_matmul_body(we_ref, w1_ref, out_ref):
    out_ref[...] = jnp.dot(we_ref[...], w1_ref[...],
                           preferred_element_type=jnp.float32
                           ).astype(jnp.bfloat16)


def _we_matmul(wordemb, W1_side):
    blk = 2000
    return pl.pallas_call(
        _we_matmul_body,
        grid=(V // blk,),
        in_specs=[
            pl.BlockSpec((blk, D_IN), lambda i: (i, 0)),
            pl.BlockSpec((D_IN, D_HID), lambda i: (0, 0)),
        ],
        out_specs=pl.BlockSpec((blk, D_HID), lambda i: (i, 0)),
        out_shape=jax.ShapeDtypeStruct((V, D_HID), jnp.bfloat16),
        name="tc_we_matmul",
    )(wordemb, W1_side)


def _scales_body(cnt_ref, sc_ref):
    cnt = cnt_ref[...]
    sc4 = lax.rsqrt(jnp.maximum(cnt[0] + cnt[1], 1.0))
    sc_ref[...] = jnp.transpose(sc4, (1, 0))


def _scales(cnt, name):
    return pl.pallas_call(
        _scales_body,
        in_specs=[pl.BlockSpec((NC, 4, NQ), lambda: (0, 0, 0))],
        out_specs=pl.BlockSpec((NQ, 4), lambda: (0, 0)),
        out_shape=jax.ShapeDtypeStruct((NQ, 4), jnp.float32),
        name=name,
    )(cnt)


def _scale_body(g1_ref, sc_ref, y_ref):
    i = pl.program_id(0)
    sc0 = sc_ref[...][:, 0:1]
    y_ref[...] = (g1_ref[...].astype(jnp.float32) * sc0).astype(jnp.bfloat16)


def _scale(g1, scales, name):
    return pl.pallas_call(
        _scale_body,
        grid=(NRB,),
        in_specs=[
            pl.BlockSpec((RB, D_HID), lambda i: (i, 0)),
            pl.BlockSpec((RB, 4), lambda i: (i, 0)),
        ],
        out_specs=pl.BlockSpec((RB, D_HID), lambda i: (i, 0)),
        out_shape=jax.ShapeDtypeStruct((NQ, D_HID), jnp.bfloat16),
        name=name,
    )(g1, scales)


def _mid_body(ag_ref, sc_ref, b1s_ref, b1u_ref, w2_ref, y_ref):
    sc = sc_ref[...]
    b = (b1s_ref[...] + b1u_ref[...])[None, :]
    w2 = w2_ref[...]
    ag = (ag_ref[0].astype(jnp.float32) + ag_ref[1].astype(jnp.float32))
    h1 = jnp.maximum(ag * sc[:, 1:2] + b, 0.0)
    y2 = jnp.dot(h1, w2, preferred_element_type=jnp.float32)
    y_ref[...] = (y2 * sc[:, 2:3]).astype(jnp.bfloat16)


def _mid(agg, scales, b1_side, b1_upd, W2_upd, name):
    return pl.pallas_call(
        _mid_body,
        grid=(NRB,),
        in_specs=[
            pl.BlockSpec((NC, RB, D_HID), lambda i: (0, i, 0)),
            pl.BlockSpec((RB, 4), lambda i: (i, 0)),
            pl.BlockSpec((D_HID,), lambda i: (0,)),
            pl.BlockSpec((D_HID,), lambda i: (0,)),
            pl.BlockSpec((D_HID, D_OUT), lambda i: (0, 0)),
        ],
        out_specs=pl.BlockSpec((RB, D_HID), lambda i: (i, 0)),
        out_shape=jax.ShapeDtypeStruct((NQ, D_HID), jnp.bfloat16),
        name=name,
    )(agg, scales, b1_side, b1_upd, W2_upd)


def _final_body(aga_ref, agp_ref, sca_ref, scp_ref, gida_ref, gidp_ref,
                b2_ref, click_ref, W0_ref, b0_ref, W1_ref, b1_ref, out_ref,
                acc_a, cnt_a, acc_p, cnt_p):
    i = pl.program_id(0)

    @pl.when(i == 0)
    def _():
        acc_a[...] = jnp.zeros((B, D_OUT), jnp.float32)
        cnt_a[...] = jnp.zeros((B, 1), jnp.float32)
        acc_p[...] = jnp.zeros((B, D_OUT), jnp.float32)
        cnt_p[...] = jnp.zeros((B, 1), jnp.float32)

    b2 = b2_ref[...][None, :]
    iota = lax.broadcasted_iota(jnp.int32, (B, RB), 0)
    ones_col = jnp.ones((RB, 1), jnp.float32)
    for ag_ref, sc_ref, gid_ref, acc, cnt in (
            (aga_ref, sca_ref, gida_ref, acc_a, cnt_a),
            (agp_ref, scp_ref, gidp_ref, acc_p, cnt_p)):
        ag = (ag_ref[0].astype(jnp.float32) + ag_ref[1].astype(jnp.float32))
        h2 = ag * sc_ref[...][:, 3:4] + b2
        gid = gid_ref[0, 0]
        mask = (gid[None, :] == iota).astype(jnp.float32)
        acc[...] += jnp.dot(mask, h2, preferred_element_type=jnp.float32)
        cnt[...] += jnp.dot(mask, ones_col, preferred_element_type=jnp.float32)

    @pl.when(i == NRB - 1)
    def _():
        anchor = acc_a[...] / jnp.maximum(cnt_a[...], 1.0)
        pos = acc_p[...] / jnp.maximum(cnt_p[...], 1.0)
        W0a = W0_ref[:D_OUT, :]
        W0b = W0_ref[D_OUT:, :]
        b0 = b0_ref[...][None, :]
        W1 = W1_ref[...]
        b1 = b1_ref[...][None, :]
        z1 = jnp.maximum(anchor @ W0a + pos @ W0b + b0, 0.0)
        s1 = jax.nn.sigmoid(z1 @ W1 + b1)
        Aa = anchor @ W0a
        Ab = anchor @ W0b
        z2 = jnp.maximum(Aa[:, None, :] + Ab[None, :, :] + b0[None, :, :], 0.0)
        s2_ = jax.nn.sigmoid(
            z2.reshape(B * B, D_OUT) @ W1 + b1).reshape(B, B)
        res = s2_ * click_ref[...].astype(jnp.float32)
        s2 = jnp.max(res, axis=1)
        eps = 1e-12
        p1 = jnp.clip(s1[:, 0], eps, 1.0 - eps)
        p2 = jnp.clip(s2, eps, 1.0 - eps)
        loss = -jnp.mean(jnp.log(p1)) - jnp.mean(jnp.log(1.0 - p2))
        out_ref[...] = jnp.reshape(loss, (1, 1))


def _final(agg2_a, agg2_p, sc_a, sc_p, gid_a, gid_p, b2_upd, click,
           D_W0, D_b0, D_W1, D_b1):
    gid_a3 = gid_a.reshape(NRB, 1, RB)
    gid_p3 = gid_p.reshape(NRB, 1, RB)
    return pl.pallas_call(
        _final_body,
        grid=(NRB,),
        in_specs=[pl.BlockSpec((NC, RB, D_HID), lambda i: (0, i, 0))] * 2 + [
            pl.BlockSpec((RB, 4), lambda i: (i, 0)),
            pl.BlockSpec((RB, 4), lambda i: (i, 0)),
            pl.BlockSpec((1, 1, RB), lambda i: (i, 0, 0)),
            pl.BlockSpec((1, 1, RB), lambda i: (i, 0, 0)),
            pl.BlockSpec((D_OUT,), lambda i: (0,)),
            pl.BlockSpec((B, B), lambda i: (0, 0)),
            pl.BlockSpec((2 * D_OUT, 32), lambda i: (0, 0)),
            pl.BlockSpec((32,), lambda i: (0,)),
            pl.BlockSpec((32, 1), lambda i: (0, 0)),
            pl.BlockSpec((1,), lambda i: (0,)),
        ],
        out_specs=pl.BlockSpec((1, 1), lambda i: (0, 0)),
        out_shape=jax.ShapeDtypeStruct((1, 1), jnp.float32),
        scratch_shapes=[
            pltpu.VMEM((B, D_OUT), jnp.float32),
            pltpu.VMEM((B, 1), jnp.float32),
            pltpu.VMEM((B, D_OUT), jnp.float32),
            pltpu.VMEM((B, 1), jnp.float32),
        ],
        name="tc_final",
    )(agg2_a, agg2_p, sc_a, sc_p, gid_a3, gid_p3, b2_upd, click,
      D_W0, D_b0, D_W1, D_b1)


def kernel(word_id_a, img_emb_a, edge_side_a, edge_upd_a, query_gid_a, word_id_p, img_emb_p, edge_side_p, edge_upd_p, query_gid_p, click_reverse, wordemb, trans_W, trans_b, W1_side, b1_side, W1_upd, b1_upd, W2_upd, b2_upd, D_W0, D_b0, D_W1, D_b1):
    pad_ids = jnp.arange(NQG - NQ, dtype=jnp.int32) % V
    wid_a = jnp.concatenate([word_id_a, pad_ids]).reshape(G_ROWS, CH)
    wid_p = jnp.concatenate([word_id_p, pad_ids]).reshape(G_ROWS, CH)
    es_a = edge_side_a.reshape(2, ROWS_E, CH)
    eu_a = edge_upd_a.reshape(2, ROWS_E, CH)
    es_p = edge_side_p.reshape(2, ROWS_E, CH)
    eu_p = edge_upd_p.reshape(2, ROWS_E, CH)

    we = _we_matmul(wordemb, W1_side)
    g1a, cnt_a = _sc_pre(es_a, eu_a, wid_a, we, "sc_pre_a")
    g1p, cnt_p = _sc_pre(es_p, eu_p, wid_p, we, "sc_pre_p")
    sc_a = _scales(cnt_a, "tc_scales_a")
    sc_p = _scales(cnt_p, "tc_scales_p")
    ya = _scale(g1a, sc_a, "tc_scale_a")
    yp = _scale(g1p, sc_p, "tc_scale_p")
    agg_a = _sc_edge(ya, es_a, "sc_edge1_a")
    agg_p = _sc_edge(yp, es_p, "sc_edge1_p")
    y2a = _mid(agg_a, sc_a, b1_side, b1_upd, W2_upd, "tc_mid_a")
    y2p = _mid(agg_p, sc_p, b1_side, b1_upd, W2_upd, "tc_mid_p")
    agg2_a = _sc_edge(y2a, eu_a, "sc_edge2_a")
    agg2_p = _sc_edge(y2p, eu_p, "sc_edge2_p")
    loss = _final(agg2_a, agg2_p, sc_a, sc_p, query_gid_a, query_gid_p,
                  b2_upd, click_reverse, D_W0, D_b0, D_W1, D_b1)
    return jnp.reshape(loss, ())
